# baseline (speedup 1.0000x reference)
import functools

import jax
import jax.numpy as jnp
from jax import lax
from jax.experimental import pallas as pl
from jax.experimental.pallas import tpu as pltpu
from jax.experimental.pallas import tpu_sc as plsc

N = 10000
E = 320000
D = 128
NC = 2
NS = 16
NW = NC * NS
C = 128
NCHUNK = 79
EPP = NCHUNK * C
RB = 3
IB = 4
ZBLK = N // C
ZMAXT = (ZBLK + NS) // NS


def _lane_bcast(v16, lane):
    idx = jnp.full((16, 1), lane, jnp.int32)
    return lax.gather(
        v16, idx,
        lax.GatherDimensionNumbers(
            offset_dims=(), collapsed_slice_dims=(0,), start_index_map=(0,)
        ),
        slice_sizes=(1,),
        mode=lax.GatherScatterMode.PROMISE_IN_BOUNDS,
    )


_mesh = plsc.VectorSubcoreMesh(
    core_axis_name="c", subcore_axis_name="s", num_cores=NC, num_subcores=NS
)


@functools.partial(
    pl.kernel,
    out_type=jax.ShapeDtypeStruct((NC, N, D), jnp.float32),
    mesh=_mesh,
    scratch_types=[
        pltpu.VMEM_SHARED((N, D), jnp.float32),
        pltpu.VMEM((IB, 2, C), jnp.int32),
        pltpu.VMEM((IB, C), jnp.float32),
        pltpu.VMEM((RB, C, D), jnp.float32),
        [pltpu.SemaphoreType.DMA] * RB,
        [pltpu.SemaphoreType.DMA] * RB,
        [pltpu.SemaphoreType.DMA] * IB,
        [pltpu.SemaphoreType.DMA] * IB,
    ],
)
def _sc_aggregate(emb_hbm, idx_hbm, w_hbm, out_hbm,
                  agg, iring, wring, rows,
                  gsems, ssems, isems, wisems):
    cid = lax.axis_index("c")
    sid = lax.axis_index("s")
    wid = cid * NS + sid
    ebase = wid * EPP

    cbase = wid * NCHUNK

    ebase = wid * EPP

    def issue_idx(j, b):
        pltpu.async_copy(idx_hbm.at[cbase + j], iring.at[b], isems[b])
        pltpu.async_copy(
            w_hbm.at[pl.ds(ebase + j * C, C)], wring.at[b], wisems[b]
        )

    def wait_idx(j, b):
        pltpu.make_async_copy(
            idx_hbm.at[cbase + j], iring.at[b], isems[b]
        ).wait()
        pltpu.make_async_copy(
            w_hbm.at[pl.ds(ebase + j * C, C)], wring.at[b], wisems[b]
        ).wait()

    def issue_gather(bi, br):
        pltpu.async_copy(emb_hbm.at[iring.at[bi, 0]], rows.at[br], gsems[br])

    def wait_gather(bi, br):
        pltpu.make_async_copy(
            emb_hbm.at[iring.at[bi, 0]], rows.at[br], gsems[br]
        ).wait()

    def issue_scatter(bi, br):
        pltpu.async_copy(
            rows.at[br], agg.at[iring.at[bi, 1]], ssems[br], add=True
        )

    def wait_scatter(bi, br):
        pltpu.make_async_copy(
            rows.at[br], agg.at[iring.at[bi, 1]], ssems[br]
        ).wait()

    def compute(bi, br):
        rows_b = rows.at[br]

        def group_body(g, _):
            w16 = wring[bi, pl.ds(g * 16, 16)]
            for i in range(16):
                wb = _lane_bcast(w16, i)
                e = g * 16 + i
                for k in range(D // 16):
                    rows_b[e, pl.ds(k * 16, 16)] = (
                        rows_b[e, pl.ds(k * 16, 16)] * wb
                    )
            return 0

        lax.fori_loop(0, C // 16, group_body, 0)

    def step(j, u):
        br = u % RB
        bi = u % IB
        wait_gather(bi, br)
        compute(bi, br)
        issue_scatter(bi, br)

        @pl.when(j >= 1)
        def _():
            wait_scatter((u - 1) % IB, (u - 1) % RB)

        @pl.when(j + 3 < NCHUNK)
        def _():
            issue_idx(j + 3, (u + 3) % IB)

        @pl.when(j + 2 < NCHUNK)
        def _():
            wait_idx(j + 2, (u + 2) % IB)
            issue_gather((u + 2) % IB, (u + 2) % RB)

    zeros16 = jnp.zeros((16,), jnp.float32)
    zbuf = rows.at[0]

    def zero_row(r, _):
        for k in range(D // 16):
            zbuf[r, pl.ds(k * 16, 16)] = zeros16
        return 0

    lax.fori_loop(0, C, zero_row, 0)
    for t in range(ZMAXT):
        blk = sid + t * NS

        @pl.when(blk < ZBLK)
        def _():
            pltpu.sync_copy(zbuf, agg.at[pl.ds(blk * C, C)])

        @pl.when(blk == ZBLK)
        def _():
            pltpu.sync_copy(zbuf.at[pl.ds(0, 16)], agg.at[pl.ds(ZBLK * C, 16)])

    plsc.subcore_barrier()

    issue_idx(0, 0)
    issue_idx(1, 1)
    issue_idx(2, 2)
    wait_idx(0, 0)
    issue_gather(0, 0)
    wait_idx(1, 1)
    issue_gather(1, 1)

    UNROLL = 12

    def tbody(t, _):
        for u in range(UNROLL):
            step(t * UNROLL + u, u)
        return 0

    lax.fori_loop(0, NCHUNK // UNROLL, tbody, 0)
    for j in range((NCHUNK // UNROLL) * UNROLL, NCHUNK):
        step(j, j)
    wait_scatter((NCHUNK - 1) % IB, (NCHUNK - 1) % RB)
    plsc.subcore_barrier()

    for t in range(ZMAXT):
        blk = sid + t * NS

        @pl.when(blk < ZBLK)
        def _():
            pltpu.sync_copy(agg.at[pl.ds(blk * C, C)], zbuf)
            pltpu.sync_copy(zbuf, out_hbm.at[cid, pl.ds(blk * C, C)])

        @pl.when(blk == ZBLK)
        def _():
            pltpu.sync_copy(agg.at[pl.ds(ZBLK * C, 16)], zbuf.at[pl.ds(0, 16)])
            pltpu.sync_copy(
                zbuf.at[pl.ds(0, 16)], out_hbm.at[cid, pl.ds(ZBLK * C, 16)]
            )


def _mm_body(p_ref, w_ref, o_ref):
    a = p_ref[0] + p_ref[1]
    o_ref[...] = lax.dot_general(
        a, w_ref[...], (((1,), (1,)), ((), ())),
        preferred_element_type=jnp.float32,
    )


_MM_BLK = 1000


def _tc_linear(partials, W):
    return pl.pallas_call(
        _mm_body,
        grid=(N // _MM_BLK,),
        in_specs=[
            pl.BlockSpec((NC, _MM_BLK, D), lambda i: (0, i, 0)),
            pl.BlockSpec((D, D), lambda i: (0, 0)),
        ],
        out_specs=pl.BlockSpec((_MM_BLK, D), lambda i: (i, 0)),
        out_shape=jax.ShapeDtypeStruct((N, D), jnp.float32),
    )(partials, W)


def _pad_worker_major(x):
    x2 = x.reshape(NW, E // NW)
    return jnp.pad(x2, ((0, 0), (0, EPP - E // NW))).reshape(NW, NCHUNK, C)


def kernel(node_emb, edge_index, edge_weight, W):
    src = _pad_worker_major(edge_index[0].astype(jnp.int32))
    dst = _pad_worker_major(edge_index[1].astype(jnp.int32))
    w = _pad_worker_major(edge_weight).reshape(NW * EPP)
    idx = jnp.stack([src, dst], axis=2).reshape(NW * NCHUNK, 2, C)
    partials = _sc_aggregate(node_emb, idx, w)
    return _tc_linear(partials, W)


# speedup vs baseline: 1.0114x; 1.0114x over previous
import functools

import jax
import jax.numpy as jnp
from jax import lax
from jax.experimental import pallas as pl
from jax.experimental.pallas import tpu as pltpu
from jax.experimental.pallas import tpu_sc as plsc

N = 10000
E = 320000
D = 128
NC = 2
NS = 16
NW = NC * NS
C = 128
NCHUNK = 79
EPP = NCHUNK * C
RB = 3
IB = 4
ZBLK = N // C
ZMAXT = (ZBLK + NS) // NS


def _lane_bcast(v16, lane):
    idx = jnp.full((16, 1), lane, jnp.int32)
    return lax.gather(
        v16, idx,
        lax.GatherDimensionNumbers(
            offset_dims=(), collapsed_slice_dims=(0,), start_index_map=(0,)
        ),
        slice_sizes=(1,),
        mode=lax.GatherScatterMode.PROMISE_IN_BOUNDS,
    )


_mesh = plsc.VectorSubcoreMesh(
    core_axis_name="c", subcore_axis_name="s", num_cores=NC, num_subcores=NS
)


@functools.partial(
    pl.kernel,
    out_type=jax.ShapeDtypeStruct((NC, N, D), jnp.float32),
    mesh=_mesh,
    scratch_types=[
        pltpu.VMEM_SHARED((N, D), jnp.float32),
        pltpu.VMEM((IB, C), jnp.int32),
        pltpu.VMEM((IB, C), jnp.int32),
        pltpu.VMEM((IB, C), jnp.float32),
        pltpu.VMEM((RB, C, D), jnp.float32),
        [pltpu.SemaphoreType.DMA] * RB,
        [pltpu.SemaphoreType.DMA] * RB,
        [pltpu.SemaphoreType.DMA] * IB,
        [pltpu.SemaphoreType.DMA] * IB,
        [pltpu.SemaphoreType.DMA] * IB,
    ],
)
def _sc_aggregate(emb_hbm, src_hbm, dst_hbm, w_hbm, out_hbm,
                  agg, sring, dring, wring, rows,
                  gsems, ssems, sisems, disems, wisems):
    cid = lax.axis_index("c")
    sid = lax.axis_index("s")
    wid = cid * NS + sid
    ebase = wid * EPP

    def issue_idx(j, b):
        sl = pl.ds(ebase + j * C, C)
        pltpu.async_copy(src_hbm.at[sl], sring.at[b], sisems[b])
        pltpu.async_copy(dst_hbm.at[sl], dring.at[b], disems[b])
        pltpu.async_copy(w_hbm.at[sl], wring.at[b], wisems[b])

    def wait_idx(j, b):
        sl = pl.ds(ebase + j * C, C)
        pltpu.make_async_copy(src_hbm.at[sl], sring.at[b], sisems[b]).wait()
        pltpu.make_async_copy(dst_hbm.at[sl], dring.at[b], disems[b]).wait()
        pltpu.make_async_copy(w_hbm.at[sl], wring.at[b], wisems[b]).wait()

    def issue_gather(bi, br):
        pltpu.async_copy(emb_hbm.at[sring.at[bi]], rows.at[br], gsems[br])

    def wait_gather(bi, br):
        pltpu.make_async_copy(
            emb_hbm.at[sring.at[bi]], rows.at[br], gsems[br]
        ).wait()

    def issue_scatter(bi, br):
        pltpu.async_copy(rows.at[br], agg.at[dring.at[bi]], ssems[br], add=True)

    def wait_scatter(bi, br):
        pltpu.make_async_copy(
            rows.at[br], agg.at[dring.at[bi]], ssems[br]
        ).wait()

    def compute(bi, br):
        rows_b = rows.at[br]

        def group_body(g, _):
            w16 = wring[bi, pl.ds(g * 16, 16)]
            for i in range(16):
                wb = _lane_bcast(w16, i)
                e = g * 16 + i
                for k in range(D // 16):
                    rows_b[e, pl.ds(k * 16, 16)] = (
                        rows_b[e, pl.ds(k * 16, 16)] * wb
                    )
            return 0

        lax.fori_loop(0, C // 16, group_body, 0)

    def step(j, u):
        br = u % RB
        bi = u % IB
        wait_gather(bi, br)
        compute(bi, br)
        issue_scatter(bi, br)

        @pl.when(j >= 1)
        def _():
            wait_scatter((u - 1) % IB, (u - 1) % RB)

        @pl.when(j + 3 < NCHUNK)
        def _():
            issue_idx(j + 3, (u + 3) % IB)

        @pl.when(j + 2 < NCHUNK)
        def _():
            wait_idx(j + 2, (u + 2) % IB)
            issue_gather((u + 2) % IB, (u + 2) % RB)

    zeros16 = jnp.zeros((16,), jnp.float32)
    zbuf = rows.at[0]

    def zero_row(r, _):
        for k in range(D // 16):
            zbuf[r, pl.ds(k * 16, 16)] = zeros16
        return 0

    lax.fori_loop(0, C, zero_row, 0)
    for t in range(ZMAXT):
        blk = sid + t * NS

        @pl.when(blk < ZBLK)
        def _():
            pltpu.sync_copy(zbuf, agg.at[pl.ds(blk * C, C)])

        @pl.when(blk == ZBLK)
        def _():
            pltpu.sync_copy(zbuf.at[pl.ds(0, 16)], agg.at[pl.ds(ZBLK * C, 16)])

    plsc.subcore_barrier()

    issue_idx(0, 0)
    issue_idx(1, 1)
    issue_idx(2, 2)
    wait_idx(0, 0)
    issue_gather(0, 0)
    wait_idx(1, 1)
    issue_gather(1, 1)

    UNROLL = 12

    def tbody(t, _):
        for u in range(UNROLL):
            step(t * UNROLL + u, u)
        return 0

    lax.fori_loop(0, NCHUNK // UNROLL, tbody, 0)
    for j in range((NCHUNK // UNROLL) * UNROLL, NCHUNK):
        step(j, j)
    wait_scatter((NCHUNK - 1) % IB, (NCHUNK - 1) % RB)
    plsc.subcore_barrier()

    for t in range(ZMAXT):
        blk = sid + t * NS

        @pl.when(blk < ZBLK)
        def _():
            pltpu.sync_copy(agg.at[pl.ds(blk * C, C)], zbuf)
            pltpu.sync_copy(zbuf, out_hbm.at[cid, pl.ds(blk * C, C)])

        @pl.when(blk == ZBLK)
        def _():
            pltpu.sync_copy(agg.at[pl.ds(ZBLK * C, 16)], zbuf.at[pl.ds(0, 16)])
            pltpu.sync_copy(
                zbuf.at[pl.ds(0, 16)], out_hbm.at[cid, pl.ds(ZBLK * C, 16)]
            )


def _mm_body(p_ref, w_ref, o_ref):
    a = p_ref[0] + p_ref[1]
    o_ref[...] = lax.dot_general(
        a, w_ref[...], (((1,), (1,)), ((), ())),
        preferred_element_type=jnp.float32,
    )


_MM_BLK = 1000


def _tc_linear(partials, W):
    return pl.pallas_call(
        _mm_body,
        grid=(N // _MM_BLK,),
        in_specs=[
            pl.BlockSpec((NC, _MM_BLK, D), lambda i: (0, i, 0)),
            pl.BlockSpec((D, D), lambda i: (0, 0)),
        ],
        out_specs=pl.BlockSpec((_MM_BLK, D), lambda i: (i, 0)),
        out_shape=jax.ShapeDtypeStruct((N, D), jnp.float32),
    )(partials, W)


def _pad_worker_major(x):
    x2 = x.reshape(NW, E // NW)
    return jnp.pad(x2, ((0, 0), (0, EPP - E // NW))).reshape(NW * EPP)


def kernel(node_emb, edge_index, edge_weight, W):
    src = _pad_worker_major(edge_index[0].astype(jnp.int32))
    dst = _pad_worker_major(edge_index[1].astype(jnp.int32))
    w = _pad_worker_major(edge_weight)
    partials = _sc_aggregate(node_emb, src, dst, w)
    return _tc_linear(partials, W)


# speedup vs baseline: 1.0140x; 1.0026x over previous
import functools

import jax
import jax.numpy as jnp
from jax import lax
from jax.experimental import pallas as pl
from jax.experimental.pallas import tpu as pltpu
from jax.experimental.pallas import tpu_sc as plsc

N = 10000
E = 320000
D = 128
NC = 2
NS = 16
NW = NC * NS
C = 128
NCHUNK = 79
EPP = NCHUNK * C
RB = 3
IB = 4
ZBLK = N // C
ZMAXT = (ZBLK + NS) // NS


def _lane_bcast(v16, lane):
    idx = jnp.full((16, 1), lane, jnp.int32)
    return lax.gather(
        v16, idx,
        lax.GatherDimensionNumbers(
            offset_dims=(), collapsed_slice_dims=(0,), start_index_map=(0,)
        ),
        slice_sizes=(1,),
        mode=lax.GatherScatterMode.PROMISE_IN_BOUNDS,
    )


_mesh = plsc.VectorSubcoreMesh(
    core_axis_name="c", subcore_axis_name="s", num_cores=NC, num_subcores=NS
)


@functools.partial(
    pl.kernel,
    out_type=jax.ShapeDtypeStruct((NC, N, D), jnp.float32),
    mesh=_mesh,
    scratch_types=[
        pltpu.VMEM_SHARED((N, D), jnp.float32),
        pltpu.VMEM((IB, C), jnp.int32),
        pltpu.VMEM((IB, C), jnp.int32),
        pltpu.VMEM((IB, C), jnp.float32),
        pltpu.VMEM((RB, C, D), jnp.float32),
        [pltpu.SemaphoreType.DMA] * RB,
        [pltpu.SemaphoreType.DMA] * RB,
        [pltpu.SemaphoreType.DMA] * IB,
        [pltpu.SemaphoreType.DMA] * IB,
        [pltpu.SemaphoreType.DMA] * IB,
    ],
)
def _sc_aggregate(emb_hbm, src_hbm, dst_hbm, w_hbm, out_hbm,
                  agg, sring, dring, wring, rows,
                  gsems, ssems, sisems, disems, wisems):
    cid = lax.axis_index("c")
    sid = lax.axis_index("s")
    wid = cid * NS + sid
    ebase = wid * EPP

    def issue_idx(j, b):
        sl = pl.ds(ebase + j * C, C)
        pltpu.async_copy(src_hbm.at[sl], sring.at[b], sisems[b])
        pltpu.async_copy(dst_hbm.at[sl], dring.at[b], disems[b])
        pltpu.async_copy(w_hbm.at[sl], wring.at[b], wisems[b])

    def wait_idx(j, b):
        sl = pl.ds(ebase + j * C, C)
        pltpu.make_async_copy(src_hbm.at[sl], sring.at[b], sisems[b]).wait()
        pltpu.make_async_copy(dst_hbm.at[sl], dring.at[b], disems[b]).wait()
        pltpu.make_async_copy(w_hbm.at[sl], wring.at[b], wisems[b]).wait()

    def issue_gather(bi, br):
        pltpu.async_copy(emb_hbm.at[sring.at[bi]], rows.at[br], gsems[br])

    def wait_gather(bi, br):
        pltpu.make_async_copy(
            emb_hbm.at[sring.at[bi]], rows.at[br], gsems[br]
        ).wait()

    def issue_scatter(bi, br):
        pltpu.async_copy(rows.at[br], agg.at[dring.at[bi]], ssems[br], add=True)

    def wait_scatter(bi, br):
        pltpu.make_async_copy(
            rows.at[br], agg.at[dring.at[bi]], ssems[br]
        ).wait()

    def compute(bi, br):
        rows_b = rows.at[br]

        def group_body(g, _):
            w16 = wring[bi, pl.ds(g * 16, 16)]
            for i in range(16):
                wb = _lane_bcast(w16, i)
                e = g * 16 + i
                for k in range(D // 16):
                    rows_b[e, pl.ds(k * 16, 16)] = (
                        rows_b[e, pl.ds(k * 16, 16)] * wb
                    )
            return 0

        lax.fori_loop(0, C // 16, group_body, 0)

    def step(j, u):
        br = u % RB
        bi = u % IB
        wait_gather(bi, br)
        compute(bi, br)
        issue_scatter(bi, br)

        @pl.when(j >= 1)
        def _():
            wait_scatter((u - 1) % IB, (u - 1) % RB)

        @pl.when(j + 3 < NCHUNK)
        def _():
            issue_idx(j + 3, (u + 3) % IB)

        @pl.when(j + 2 < NCHUNK)
        def _():
            wait_idx(j + 2, (u + 2) % IB)
            issue_gather((u + 2) % IB, (u + 2) % RB)

    zeros16 = jnp.zeros((16,), jnp.float32)
    zbuf = rows.at[0]

    def zero_row(r, _):
        for k in range(D // 16):
            zbuf[r, pl.ds(k * 16, 16)] = zeros16
        return 0

    lax.fori_loop(0, C, zero_row, 0)
    for t in range(ZMAXT):
        blk = sid + t * NS

        @pl.when(blk < ZBLK)
        def _():
            pltpu.sync_copy(zbuf, agg.at[pl.ds(blk * C, C)])

        @pl.when(blk == ZBLK)
        def _():
            pltpu.sync_copy(zbuf.at[pl.ds(0, 16)], agg.at[pl.ds(ZBLK * C, 16)])

    plsc.subcore_barrier()

    issue_idx(0, 0)
    issue_idx(1, 1)
    issue_idx(2, 2)
    wait_idx(0, 0)
    issue_gather(0, 0)
    wait_idx(1, 1)
    issue_gather(1, 1)

    UNROLL = 12

    def tbody(t, _):
        for u in range(UNROLL):
            step(t * UNROLL + u, u)
        return 0

    lax.fori_loop(0, NCHUNK // UNROLL, tbody, 0)
    for j in range((NCHUNK // UNROLL) * UNROLL, NCHUNK):
        step(j, j)
    wait_scatter((NCHUNK - 1) % IB, (NCHUNK - 1) % RB)
    plsc.subcore_barrier()

    for t in range(ZMAXT):
        blk = sid + t * NS

        @pl.when(blk < ZBLK)
        def _():
            pltpu.sync_copy(
                agg.at[pl.ds(blk * C, C)], out_hbm.at[cid, pl.ds(blk * C, C)]
            )

        @pl.when(blk == ZBLK)
        def _():
            pltpu.sync_copy(
                agg.at[pl.ds(ZBLK * C, 16)],
                out_hbm.at[cid, pl.ds(ZBLK * C, 16)],
            )


def _mm_body(p_ref, w_ref, o_ref):
    a = p_ref[0] + p_ref[1]
    o_ref[...] = lax.dot_general(
        a, w_ref[...], (((1,), (1,)), ((), ())),
        preferred_element_type=jnp.float32,
    )


_MM_BLK = 1000


def _tc_linear(partials, W):
    return pl.pallas_call(
        _mm_body,
        grid=(N // _MM_BLK,),
        in_specs=[
            pl.BlockSpec((NC, _MM_BLK, D), lambda i: (0, i, 0)),
            pl.BlockSpec((D, D), lambda i: (0, 0)),
        ],
        out_specs=pl.BlockSpec((_MM_BLK, D), lambda i: (i, 0)),
        out_shape=jax.ShapeDtypeStruct((N, D), jnp.float32),
    )(partials, W)


def _pad_worker_major(x):
    x2 = x.reshape(NW, E // NW)
    return jnp.pad(x2, ((0, 0), (0, EPP - E // NW))).reshape(NW * EPP)


def kernel(node_emb, edge_index, edge_weight, W):
    src = _pad_worker_major(edge_index[0].astype(jnp.int32))
    dst = _pad_worker_major(edge_index[1].astype(jnp.int32))
    w = _pad_worker_major(edge_weight)
    partials = _sc_aggregate(node_emb, src, dst, w)
    return _tc_linear(partials, W)
